# Initial kernel scaffold; baseline (speedup 1.0000x reference)
import jax
import jax.numpy as jnp
from jax.experimental import pallas as pl


def kernel(y, A):
    raise NotImplementedError("write your pallas kernel here")


# speedup vs baseline: 4.8131x; 4.8131x over previous
import functools

import jax
import jax.numpy as jnp
from jax.experimental import pallas as pl
from jax.experimental.pallas import tpu as pltpu

_K_ITERS = 5
_S = 2048
_BN = 2048


def _iht_body(yT_ref, A_ref, out_ref, d_ref, bits_ref, aT_ref, bT_ref,
              thr_ref, *, nb, s, k_iters):
    p = pl.program_id(0)
    n = pl.program_id(1)
    is_fwd = (p % 2) == 1

    def masked_x(i):
        return jnp.where(bits_ref[i] >= thr_ref[...], d_ref[i], 0.0)

    @pl.when(is_fwd)
    def _fwd():
        @pl.when(n == 0)
        def _():
            aT_ref[...] = jnp.zeros_like(aT_ref)

        xblk = masked_x(n)
        aT_ref[...] += jax.lax.dot_general(
            A_ref[...], xblk, (((1,), (1,)), ((), ())),
            preferred_element_type=jnp.float32)

        @pl.when(n == nb - 1)
        def _():
            bT_ref[...] = yT_ref[...] - aT_ref[...]

    @pl.when(jnp.logical_not(is_fwd))
    def _bwd():
        @pl.when((p == 0) & (n == 0))
        def _():
            bT_ref[...] = yT_ref[...]

        c = jax.lax.dot_general(
            bT_ref[...], A_ref[...], (((0,), (0,)), ((), ())),
            preferred_element_type=jnp.float32)
        xblk = jnp.where(p == 0, jnp.zeros_like(c), masked_x(n))
        d = xblk + c
        d_ref[n] = d
        bits_ref[n] = jax.lax.bitcast_convert_type(jnp.abs(d), jnp.int32)

        @pl.when(n == nb - 1)
        def _thresh():
            bits = bits_ref[...]
            b_rows = bits.shape[1]

            def step(_, lohi):
                lo, hi = lohi
                mid = lo + ((hi - lo) >> 1)
                cnt = jnp.sum(
                    jnp.sum((bits >= mid[None]).astype(jnp.int32), axis=0),
                    axis=1, keepdims=True)
                ge = cnt >= s
                return jnp.where(ge, mid, lo), jnp.where(ge, hi, mid)

            lo, _ = jax.lax.fori_loop(
                0, 31, step,
                (jnp.zeros((b_rows, 1), jnp.int32),
                 jnp.full((b_rows, 1), jnp.int32(2**31 - 1))))
            thr_ref[...] = lo

            @pl.when(p == 2 * k_iters - 2)
            def _write_out():
                for i in range(nb):
                    out_ref[i] = jnp.where(bits_ref[i] >= lo, d_ref[i], 0.0)


@jax.jit
def kernel(y, A):
    B, M = y.shape
    _, N = A.shape
    nb = N // _BN
    yT = y.T

    grid = (2 * _K_ITERS - 1, nb)
    body = functools.partial(_iht_body, nb=nb, s=_S, k_iters=_K_ITERS)
    out = pl.pallas_call(
        body,
        grid=grid,
        in_specs=[
            pl.BlockSpec((M, B), lambda p, n: (0, 0)),
            pl.BlockSpec((M, _BN), lambda p, n: (0, n)),
        ],
        out_specs=pl.BlockSpec((nb, B, _BN), lambda p, n: (0, 0, 0)),
        out_shape=jax.ShapeDtypeStruct((nb, B, _BN), jnp.float32),
        scratch_shapes=[
            pltpu.VMEM((nb, B, _BN), jnp.float32),
            pltpu.VMEM((nb, B, _BN), jnp.int32),
            pltpu.VMEM((M, B), jnp.float32),
            pltpu.VMEM((M, B), jnp.float32),
            pltpu.VMEM((B, 1), jnp.int32),
        ],
    )(yT, A)
    return out.transpose(1, 0, 2).reshape(B, N)
